# baseline (speedup 1.0000x reference)
import jax
import jax.numpy as jnp
from jax import lax
from jax.experimental import pallas as pl
from jax.experimental.pallas import tpu as pltpu
from jax.experimental.pallas import tpu_sc as plsc

_NUM_CORES = 2
_NUM_SUBCORES = 16
_NW = _NUM_CORES * _NUM_SUBCORES

_D = 32
_B = 16384 * 50
_B_PER_W = _B // _NW
_NBUF = 4
_CHUNK = 800
_N_GROUPS = _B_PER_W // (_NBUF * _CHUNK)


def _gather_body(table_hbm, idx_hbm, out_hbm, idx_v, rows_v, *sems):
    sem_i = sems[0:_NBUF]
    sem_g = sems[_NBUF:2 * _NBUF]
    sem_o = sems[2 * _NBUF:3 * _NBUF]
    wid = lax.axis_index("s") * _NUM_CORES + lax.axis_index("c")
    base = wid * _B_PER_W

    def group(g, carry):
        off = base + g * _NBUF * _CHUNK
        c_i = [
            pltpu.async_copy(
                idx_hbm.at[pl.ds(off + b * _CHUNK, _CHUNK)],
                idx_v.at[b], sem_i[b])
            for b in range(_NBUF)
        ]
        c_g = []
        for b in range(_NBUF):
            c_i[b].wait()
            c_g.append(pltpu.async_copy(
                table_hbm.at[idx_v.at[b]], rows_v.at[b], sem_g[b]))
        c_o = []
        for b in range(_NBUF):
            c_g[b].wait()
            c_o.append(pltpu.async_copy(
                rows_v.at[b],
                out_hbm.at[pl.ds(off + b * _CHUNK, _CHUNK)], sem_o[b]))
        for b in range(_NBUF):
            c_o[b].wait()
        return carry

    lax.fori_loop(0, _N_GROUPS, group, 0)


@jax.jit
def _gather(token_ids_flat, embedding):
    mesh = plsc.VectorSubcoreMesh(core_axis_name="c", subcore_axis_name="s")
    run = pl.kernel(
        _gather_body,
        out_type=jax.ShapeDtypeStruct((_B, _D), jnp.float32),
        mesh=mesh,
        scratch_types=[
            pltpu.VMEM((_NBUF, _CHUNK), jnp.int32),
            pltpu.VMEM((_NBUF, _CHUNK, _D), jnp.float32),
        ] + [pltpu.SemaphoreType.DMA] * (3 * _NBUF),
        compiler_params=pltpu.CompilerParams(use_tc_tiling_on_sc=False),
    )
    return run(embedding, token_ids_flat)


def kernel(token_ids, embedding):
    flat = token_ids.reshape(-1).astype(jnp.int32)
    out = _gather(flat, embedding)
    return out.reshape(token_ids.shape + (embedding.shape[1],))


# speedup vs baseline: 1.3221x; 1.3221x over previous
import jax
import jax.numpy as jnp
from jax import lax
from jax.experimental import pallas as pl
from jax.experimental.pallas import tpu as pltpu
from jax.experimental.pallas import tpu_sc as plsc

_NUM_CORES = 2
_NUM_SUBCORES = 16
_NW = _NUM_CORES * _NUM_SUBCORES

_D = 32
_T = 16384
_S = 50
_TB = _T // _NW
_L = 16


def _gather_body(table_hbm, idx_hbm, out_hbm, idx_v, rows_v, trans_v, *sems):
    sem_i = sems[0:2]
    sem_g = sems[2:4]
    sem_o = sems[4:6]
    wid = lax.axis_index("s") * _NUM_CORES + lax.axis_index("c")
    t0 = wid * _TB

    def start_idx(s, b):
        return pltpu.async_copy(
            idx_hbm.at[s, pl.ds(t0, _TB)], idx_v.at[b], sem_i[b])

    def wait_idx(b):
        pltpu.make_async_copy(
            idx_hbm.at[0, pl.ds(t0, _TB)], idx_v.at[b], sem_i[b]).wait()

    def start_gather(b):
        return pltpu.async_copy(
            table_hbm.at[idx_v.at[b]], rows_v.at[b], sem_g[b])

    def wait_gather(b):
        pltpu.make_async_copy(
            table_hbm.at[idx_v.at[b]], rows_v.at[b], sem_g[b]).wait()

    def start_out(s, b):
        return pltpu.async_copy(
            trans_v.at[b], out_hbm.at[s, :, pl.ds(t0, _TB)], sem_o[b])

    def wait_out(b):
        pltpu.make_async_copy(
            trans_v.at[b], out_hbm.at[0, :, pl.ds(t0, _TB)], sem_o[b]).wait()

    lanes = lax.iota(jnp.int32, _L)

    def transpose(b):
        def tblock(k, carry):
            row_idx = lanes + k * _L
            for j in range(_D):
                col_idx = jnp.full((_L,), j, jnp.int32)
                v = plsc.load_gather(rows_v.at[b], [row_idx, col_idx])
                trans_v[b, j, pl.ds(k * _L, _L)] = v
            return carry

        lax.fori_loop(0, _TB // _L, tblock, 0)

    n_pairs = _S // 2

    start_idx(0, 0).wait()
    start_gather(0)
    start_idx(1, 1)

    def pair(p, carry):
        s0 = 2 * p
        wait_idx(1)
        start_gather(1)
        wait_gather(0)

        @pl.when(p > 0)
        def _():
            wait_out(0)

        transpose(0)
        start_out(s0, 0)

        @pl.when(p < n_pairs - 1)
        def _():
            start_idx(s0 + 2, 0)

        wait_gather(1)

        @pl.when(p < n_pairs - 1)
        def _():
            wait_idx(0)
            start_gather(0)

        @pl.when(p > 0)
        def _():
            wait_out(1)

        transpose(1)
        start_out(s0 + 1, 1)

        @pl.when(p < n_pairs - 1)
        def _():
            start_idx(s0 + 3, 1)

        return carry

    lax.fori_loop(0, n_pairs, pair, 0)
    wait_out(0)
    wait_out(1)


@jax.jit
def _gather(token_ids_t, embedding):
    mesh = plsc.VectorSubcoreMesh(core_axis_name="c", subcore_axis_name="s")
    run = pl.kernel(
        _gather_body,
        out_type=jax.ShapeDtypeStruct((_S, _D, _T), jnp.float32),
        mesh=mesh,
        scratch_types=[
            pltpu.VMEM((2, _TB), jnp.int32),
            pltpu.VMEM((2, _TB, _D), jnp.float32),
            pltpu.VMEM((2, _D, _TB), jnp.float32),
        ] + [pltpu.SemaphoreType.DMA] * 6,
        compiler_params=pltpu.CompilerParams(
            use_tc_tiling_on_sc=False, needs_layout_passes=False),
    )
    return run(embedding, token_ids_t)


def kernel(token_ids, embedding):
    out_t = _gather(token_ids.T.astype(jnp.int32), embedding)
    return out_t.transpose(2, 0, 1)


# speedup vs baseline: 1.7468x; 1.3212x over previous
import jax
import jax.numpy as jnp
from jax import lax
from jax.experimental import pallas as pl
from jax.experimental.pallas import tpu as pltpu
from jax.experimental.pallas import tpu_sc as plsc

_NUM_CORES = 2
_NUM_SUBCORES = 16
_NW = _NUM_CORES * _NUM_SUBCORES

_D = 32
_T = 16384
_S = 50
_TB = _T // _NW


def _gather_body(table_hbm, idx_hbm, out_hbm, idx_v, rows_v, *sems):
    sem_i = sems[0:2]
    sem_g = sems[2:4]
    sem_o = sems[4:6]
    wid = lax.axis_index("s") * _NUM_CORES + lax.axis_index("c")
    t0 = wid * _TB

    def start_idx(s, b):
        return pltpu.async_copy(
            idx_hbm.at[s, pl.ds(t0, _TB)], idx_v.at[b], sem_i[b])

    def wait_idx(b):
        pltpu.make_async_copy(
            idx_hbm.at[0, pl.ds(t0, _TB)], idx_v.at[b], sem_i[b]).wait()

    def start_gather(b):
        return pltpu.async_copy(
            table_hbm.at[idx_v.at[b]], rows_v.at[b], sem_g[b])

    def wait_gather(b):
        pltpu.make_async_copy(
            table_hbm.at[idx_v.at[b]], rows_v.at[b], sem_g[b]).wait()

    def start_out(s, b):
        return pltpu.async_copy(
            rows_v.at[b], out_hbm.at[s, pl.ds(t0, _TB)], sem_o[b])

    def wait_out(b):
        pltpu.make_async_copy(
            rows_v.at[b], out_hbm.at[0, pl.ds(t0, _TB)], sem_o[b]).wait()

    n_pairs = _S // 2

    start_idx(0, 0).wait()
    start_gather(0)
    start_idx(1, 1)

    def pair(p, carry):
        s0 = 2 * p
        wait_idx(1)

        @pl.when(p > 0)
        def _():
            wait_out(1)

        start_gather(1)
        wait_gather(0)
        start_out(s0, 0)

        @pl.when(p < n_pairs - 1)
        def _():
            start_idx(s0 + 2, 0)

        wait_gather(1)
        start_out(s0 + 1, 1)

        @pl.when(p < n_pairs - 1)
        def _():
            wait_idx(0)
            wait_out(0)
            start_gather(0)
            start_idx(s0 + 3, 1)

        return carry

    lax.fori_loop(0, n_pairs, pair, 0)
    wait_out(0)
    wait_out(1)


@jax.jit
def _gather(token_ids_t, embedding):
    mesh = plsc.VectorSubcoreMesh(core_axis_name="c", subcore_axis_name="s")
    run = pl.kernel(
        _gather_body,
        out_type=jax.ShapeDtypeStruct((_S, _T, _D), jnp.float32),
        mesh=mesh,
        scratch_types=[
            pltpu.VMEM((2, _TB), jnp.int32),
            pltpu.VMEM((2, _TB, _D), jnp.float32),
        ] + [pltpu.SemaphoreType.DMA] * 6,
        compiler_params=pltpu.CompilerParams(
            use_tc_tiling_on_sc=False, needs_layout_passes=False),
    )
    return run(embedding, token_ids_t)


def kernel(token_ids, embedding):
    out_t = _gather(token_ids.T.astype(jnp.int32), embedding)
    return out_t.transpose(1, 0, 2)


# speedup vs baseline: 1.9999x; 1.1449x over previous
import jax
import jax.numpy as jnp
from jax import lax
from jax.experimental import pallas as pl
from jax.experimental.pallas import tpu as pltpu
from jax.experimental.pallas import tpu_sc as plsc

_NUM_CORES = 2
_NUM_SUBCORES = 16
_NW = _NUM_CORES * _NUM_SUBCORES

_D = 32
_T = 16384
_S = 50
_TB = _T // _NW
_L = 16
_PAD = _TB + 17


def _gather_body(table_hbm, idx_hbm, out_hbm, idx_v, rows_v, trans_v, *sems):
    sem_i = sems[0:2]
    sem_g = sems[2:4]
    sem_o = sems[4:6]
    wid = lax.axis_index("s") * _NUM_CORES + lax.axis_index("c")
    t0 = wid * _TB

    def start_idx(s, b):
        return pltpu.async_copy(
            idx_hbm.at[s, pl.ds(t0, _TB)], idx_v.at[b], sem_i[b])

    def wait_idx(b):
        pltpu.make_async_copy(
            idx_hbm.at[0, pl.ds(t0, _TB)], idx_v.at[b], sem_i[b]).wait()

    def start_gather(b):
        return pltpu.async_copy(
            table_hbm.at[idx_v.at[b]], rows_v.at[b], sem_g[b])

    def wait_gather(b):
        pltpu.make_async_copy(
            table_hbm.at[idx_v.at[b]], rows_v.at[b], sem_g[b]).wait()

    def start_out(s, b):
        return pltpu.async_copy(
            trans_v.at[b, :, pl.ds(0, _TB)],
            out_hbm.at[s, :, pl.ds(t0, _TB)], sem_o[b])

    def wait_out(b):
        pltpu.make_async_copy(
            trans_v.at[b, :, pl.ds(0, _TB)],
            out_hbm.at[0, :, pl.ds(t0, _TB)], sem_o[b]).wait()

    lanes = lax.iota(jnp.int32, _L)
    jlo = lanes
    jhi = lanes + _L

    def transpose(b):
        def trow(t, carry):
            tcol = jnp.full((_L,), 0, jnp.int32) + t
            x0 = rows_v[b, t, pl.ds(0, _L)]
            plsc.store_scatter(trans_v.at[b], [jlo, tcol], x0)
            x1 = rows_v[b, t, pl.ds(_L, _L)]
            plsc.store_scatter(trans_v.at[b], [jhi, tcol], x1)
            return carry

        lax.fori_loop(0, _TB, trow, 0, unroll=8)

    n_pairs = _S // 2

    start_idx(0, 0).wait()
    start_gather(0)
    start_idx(1, 1)

    def pair(p, carry):
        s0 = 2 * p
        wait_idx(1)
        start_gather(1)
        wait_gather(0)

        @pl.when(p > 0)
        def _():
            wait_out(0)

        transpose(0)
        start_out(s0, 0)

        @pl.when(p < n_pairs - 1)
        def _():
            start_idx(s0 + 2, 0)

        wait_gather(1)

        @pl.when(p < n_pairs - 1)
        def _():
            wait_idx(0)
            start_gather(0)

        @pl.when(p > 0)
        def _():
            wait_out(1)

        transpose(1)
        start_out(s0 + 1, 1)

        @pl.when(p < n_pairs - 1)
        def _():
            start_idx(s0 + 3, 1)

        return carry

    lax.fori_loop(0, n_pairs, pair, 0)
    wait_out(0)
    wait_out(1)


@jax.jit
def _gather(token_ids_t, embedding):
    mesh = plsc.VectorSubcoreMesh(core_axis_name="c", subcore_axis_name="s")
    run = pl.kernel(
        _gather_body,
        out_type=jax.ShapeDtypeStruct((_S, _D, _T), jnp.float32),
        mesh=mesh,
        scratch_types=[
            pltpu.VMEM((2, _TB), jnp.int32),
            pltpu.VMEM((2, _TB, _D), jnp.float32),
            pltpu.VMEM((2, _D, _PAD), jnp.float32),
        ] + [pltpu.SemaphoreType.DMA] * 6,
        compiler_params=pltpu.CompilerParams(
            use_tc_tiling_on_sc=False, needs_layout_passes=False),
    )
    return run(embedding, token_ids_t)


def kernel(token_ids, embedding):
    out_t = _gather(token_ids.T.astype(jnp.int32), embedding)
    return out_t.transpose(2, 0, 1)
